# baseline (speedup 1.0000x reference)
import functools

import jax
import jax.numpy as jnp
from jax import lax
from jax.experimental import pallas as pl
from jax.experimental.pallas import tpu as pltpu
from jax.experimental.pallas import tpu_sc as plsc

NUM_CORES = 2
NUM_SUBCORES = 16
NUM_WORKERS = NUM_CORES * NUM_SUBCORES
HB = 8
CB = 256
BT = CB // 128
CHUNK = 128


def _gather_kernel(n_h, n_b, v, d, idx_hbm, table_hbm, out_hbm,
                   idx_v, stage_v, trans_v, table_sp, gsem, osem):
    wid = lax.axis_index("s") * NUM_CORES + lax.axis_index("c")

    @pl.when(lax.axis_index("s") == 0)
    def _():
        pltpu.sync_copy(table_hbm, table_sp)

    plsc.subcore_barrier()

    n_bu = n_b // CB
    n_units = (n_h // HB) * n_bu
    n_mine = lax.div(n_units - 1 - wid, NUM_WORKERS) + 1

    lanes = lax.iota(jnp.int32, 16)
    rm = [lax.rem(lanes + s, 16) for s in range(16)]
    rm8 = [lax.div(rm[s], 8) for s in range(16)]
    rmm8 = [lax.rem(rm[s], 8) for s in range(16)]

    def do_unit(k, carry):
        u = wid + k * NUM_WORKERS
        h0 = lax.div(u, n_bu) * HB
        bt0 = lax.rem(u, n_bu) * BT

        pltpu.sync_copy(idx_hbm.at[pl.ds(h0, HB), pl.ds(bt0 * 128, CB)],
                        idx_v)

        def gathers(hh, p, issue):
            for c in range(CB // CHUNK):
                mk = pltpu.async_copy if issue else (
                    lambda s, dst, m: pltpu.make_async_copy(s, dst, m).wait())
                mk(
                    table_sp.at[idx_v.at[hh, pl.ds(c * CHUNK, CHUNK)]],
                    stage_v.at[p, pl.ds(c * CHUNK, CHUNK)],
                    gsem,
                )

        gathers(0, 0, True)

        def hbody(hh, carry2):
            p = lax.rem(hh, 2)
            gathers(hh, p, False)

            @pl.when(hh + 1 < HB)
            def _():
                gathers(hh + 1, 1 - p, True)

            @pl.when(k * HB + hh >= 2)
            def _():
                pltpu.make_async_copy(
                    trans_v.at[p], out_hbm.at[h0, :, pl.ds(bt0, BT)],
                    osem).wait()

            @plsc.parallel_loop(0, (d // 16) * 8, unroll=2)
            def tbody(q):
                eb = lax.div(q, 8) * 16
                i16 = lax.rem(q, 8) * 16
                d3 = lanes + i16
                for b_t in range(BT):
                    rows = lanes + (b_t * 128 + i16)
                    for s in range(16):
                        col = rm[s] + eb
                        vals = plsc.load_gather(stage_v.at[p], [rows, col])
                        d0 = rm8[s] + lax.div(eb, 8)
                        d1 = jnp.full((16,), b_t, jnp.int32)
                        plsc.store_scatter(trans_v.at[p],
                                           [d0, d1, rmm8[s], d3], vals)

            pltpu.async_copy(trans_v.at[p],
                             out_hbm.at[h0 + hh, :, pl.ds(bt0, BT)], osem)
            return carry2

        lax.fori_loop(0, HB, hbody, 0)
        return carry

    lax.fori_loop(0, n_mine, do_unit, 0)

    for j in range(2):
        pltpu.make_async_copy(trans_v.at[j],
                              out_hbm.at[0, :, pl.ds(0, BT)], osem).wait()


def kernel(pos_matrix, pe):
    b, h = pos_matrix.shape
    v, d = pe.shape
    assert h % HB == 0 and b % CB == 0 and d % 8 == 0

    idx_t = pos_matrix.T.astype(jnp.int32)

    mesh = plsc.VectorSubcoreMesh(core_axis_name="c", subcore_axis_name="s")
    k = functools.partial(
        pl.kernel,
        mesh=mesh,
        out_type=jax.ShapeDtypeStruct((h, d // 8, b // 128, 8, 128),
                                      jnp.float32),
        scratch_types=[
            pltpu.VMEM((HB, CB), jnp.int32),
            pltpu.VMEM((2, CB, d), jnp.float32),
            pltpu.VMEM((2, d // 8, BT, 8, 128), jnp.float32),
            pltpu.VMEM_SHARED((v, d), jnp.float32),
            pltpu.SemaphoreType.DMA,
            pltpu.SemaphoreType.DMA,
        ],
        compiler_params=pltpu.CompilerParams(use_tc_tiling_on_sc=False,
                                             needs_layout_passes=False),
    )(functools.partial(_gather_kernel, h, b, v, d))

    x5 = k(idx_t, pe)
    return x5.transpose(2, 4, 0, 1, 3).reshape(b, h, d)


# speedup vs baseline: 1.7785x; 1.7785x over previous
import functools

import jax
import jax.numpy as jnp
from jax import lax
from jax.experimental import pallas as pl
from jax.experimental.pallas import tpu as pltpu
from jax.experimental.pallas import tpu_sc as plsc

NUM_CORES = 2
NUM_SUBCORES = 16
NUM_WORKERS = NUM_CORES * NUM_SUBCORES
HB = 8
CB = 256
BT = CB // 128
CHUNK = 128


def _gather_kernel(n_h, n_b, v, d, idx_hbm, table_hbm, out_hbm,
                   idx_v, stage_v, trans_v, table_sp, gsem, osem):
    wid = lax.axis_index("s") * NUM_CORES + lax.axis_index("c")

    @pl.when(lax.axis_index("s") == 0)
    def _():
        pltpu.sync_copy(table_hbm, table_sp)

    plsc.subcore_barrier()

    n_bu = n_b // CB
    n_units = (n_h // HB) * n_bu
    n_mine = lax.div(n_units - 1 - wid, NUM_WORKERS) + 1

    lanes = lax.iota(jnp.int32, 16)
    rm = [lax.rem(lanes + s, 16) for s in range(16)]
    rm8 = [lax.div(rm[s], 8) for s in range(16)]
    rmm8 = [lax.rem(rm[s], 8) for s in range(16)]

    def do_unit(k, carry):
        u = wid + k * NUM_WORKERS
        h0 = lax.div(u, n_bu) * HB
        bt0 = lax.rem(u, n_bu) * BT

        pltpu.sync_copy(idx_hbm.at[pl.ds(h0, HB), pl.ds(bt0 * 128, CB)],
                        idx_v)

        def gathers(hh, p, issue):
            for c in range(CB // CHUNK):
                mk = pltpu.async_copy if issue else (
                    lambda s, dst, m: pltpu.make_async_copy(s, dst, m).wait())
                mk(
                    table_sp.at[idx_v.at[hh, pl.ds(c * CHUNK, CHUNK)]],
                    stage_v.at[p, pl.ds(c * CHUNK, CHUNK)],
                    gsem,
                )

        gathers(0, 0, True)
        for hh in range(HB):
            p = hh % 2
            gathers(hh, p, False)
            if hh + 1 < HB:
                gathers(hh + 1, 1 - p, True)

            @pl.when(k * HB + hh >= 2)
            def _():
                pltpu.make_async_copy(
                    trans_v.at[p], out_hbm.at[h0, :, pl.ds(bt0, BT)],
                    osem).wait()

            @plsc.parallel_loop(0, (d // 16) * 8, unroll=1)
            def tbody(q):
                eb = lax.div(q, 8) * 16
                i16 = lax.rem(q, 8) * 16
                d3 = lanes + i16
                for b_t in range(BT):
                    rows = lanes + (b_t * 128 + i16)
                    for s in range(16):
                        col = rm[s] + eb
                        vals = plsc.load_gather(stage_v.at[p], [rows, col])
                        d0 = rm8[s] + lax.div(eb, 8)
                        d1 = jnp.full((16,), b_t, jnp.int32)
                        plsc.store_scatter(trans_v.at[p],
                                           [d0, d1, rmm8[s], d3], vals)

            pltpu.async_copy(trans_v.at[p],
                             out_hbm.at[h0 + hh, :, pl.ds(bt0, BT)], osem)
        return carry

    lax.fori_loop(0, n_mine, do_unit, 0)

    for j in range(2):
        pltpu.make_async_copy(trans_v.at[j],
                              out_hbm.at[0, :, pl.ds(0, BT)], osem).wait()


def kernel(pos_matrix, pe):
    b, h = pos_matrix.shape
    v, d = pe.shape
    assert h % HB == 0 and b % CB == 0 and d % 8 == 0

    idx_t = pos_matrix.T.astype(jnp.int32)

    mesh = plsc.VectorSubcoreMesh(core_axis_name="c", subcore_axis_name="s")
    k = functools.partial(
        pl.kernel,
        mesh=mesh,
        out_type=jax.ShapeDtypeStruct((h, d // 8, b // 128, 8, 128),
                                      jnp.float32),
        scratch_types=[
            pltpu.VMEM((HB, CB), jnp.int32),
            pltpu.VMEM((2, CB, d), jnp.float32),
            pltpu.VMEM((2, d // 8, BT, 8, 128), jnp.float32),
            pltpu.VMEM_SHARED((v, d), jnp.float32),
            pltpu.SemaphoreType.DMA,
            pltpu.SemaphoreType.DMA,
        ],
        compiler_params=pltpu.CompilerParams(use_tc_tiling_on_sc=False,
                                             needs_layout_passes=False),
    )(functools.partial(_gather_kernel, h, b, v, d))

    x5 = k(idx_t, pe)
    return x5.transpose(2, 4, 0, 1, 3).reshape(b, h, d)
